# baseline (speedup 1.0000x reference)
import functools

import jax
import jax.numpy as jnp
from jax import lax
from jax.experimental import pallas as pl
from jax.experimental.pallas import tpu as pltpu
from jax.experimental.pallas import tpu_sc as plsc

_NC = 2
_NS = 16
_NW = _NC * _NS

_SUB = 128


@functools.partial(jax.jit, static_argnames=("n", "d"))
def _sc_gather(idx_flat, table, n, d):
    v = table.shape[0]
    n_per_w = n // _NW
    n_sub = n_per_w // _SUB

    mesh = plsc.VectorSubcoreMesh(core_axis_name="c", subcore_axis_name="s")

    @functools.partial(
        pl.kernel,
        out_type=jax.ShapeDtypeStruct((n, d), jnp.float32),
        mesh=mesh,
        scratch_types=[
            pltpu.VMEM((n_per_w,), jnp.int32),
            pltpu.VMEM_SHARED((v, d), jnp.float32),
            pltpu.VMEM((_SUB, d), jnp.float32),
            pltpu.VMEM((_SUB, d), jnp.float32),
            pltpu.SemaphoreType.DMA,
            pltpu.SemaphoreType.DMA,
            pltpu.SemaphoreType.DMA,
            pltpu.SemaphoreType.DMA,
        ],
    )
    def k(idx_hbm, table_hbm, out_hbm, idx_v, table_v, rows0, rows1,
          g0, g1, w0, w1):
        wid = lax.axis_index("s") * _NC + lax.axis_index("c")
        base = wid * n_per_w

        @pl.when(lax.axis_index("s") == 0)
        def _():
            pltpu.sync_copy(table_hbm, table_v)

        plsc.subcore_barrier()
        pltpu.sync_copy(idx_hbm.at[pl.ds(base, n_per_w)], idx_v)

        def gather_start(j, rows, sem):
            pltpu.async_copy(table_v.at[idx_v.at[pl.ds(j * _SUB, _SUB)]], rows, sem)

        def gather_wait(rows, sem):
            pltpu.make_async_copy(table_v.at[idx_v.at[pl.ds(0, _SUB)]], rows, sem).wait()

        def write_start(j, rows, sem):
            pltpu.async_copy(rows, out_hbm.at[pl.ds(base + j * _SUB, _SUB)], sem)

        def write_wait(rows, sem):
            pltpu.make_async_copy(rows, out_hbm.at[pl.ds(base, _SUB)], sem).wait()

        gather_start(0, rows0, g0)
        n_pair = n_sub // 2

        def pair(i, carry):
            c = 2 * i

            @pl.when(i > 0)
            def _():
                write_wait(rows1, w1)

            gather_start(c + 1, rows1, g1)
            gather_wait(rows0, g0)
            write_start(c, rows0, w0)

            write_wait(rows0, w0)

            @pl.when(i + 1 < n_pair)
            def _():
                gather_start(c + 2, rows0, g0)

            gather_wait(rows1, g1)
            write_start(c + 1, rows1, w1)
            return carry

        lax.fori_loop(0, n_pair, pair, 0)
        write_wait(rows1, w1)

    return k(idx_flat, table)


@functools.partial(jax.jit, static_argnames=("n", "d"))
def _tc_onehot(idx_flat, table, n, d):
    blk = 2048
    tbl = jnp.zeros((128, d), table.dtype).at[: table.shape[0]].set(table)
    idx2 = idx_flat.reshape(n, 1)

    def body(idx_ref, tbl_ref, out_ref):
        oh = (idx_ref[...] == lax.broadcasted_iota(jnp.int32, (blk, 128), 1))
        out_ref[...] = jnp.dot(oh.astype(jnp.float32), tbl_ref[...],
                               preferred_element_type=jnp.float32)

    return pl.pallas_call(
        body,
        grid=(n // blk,),
        in_specs=[pl.BlockSpec((blk, 1), lambda i: (i, 0)),
                  pl.BlockSpec((128, d), lambda i: (0, 0))],
        out_specs=pl.BlockSpec((blk, d), lambda i: (i, 0)),
        out_shape=jax.ShapeDtypeStruct((n, d), jnp.float32),
    )(idx2, tbl)


def kernel(actions, embed_weight):
    b, a, l = actions.shape
    v, d = embed_weight.shape
    n = b * a * l
    idx_flat = actions.reshape(n)
    out = _tc_onehot(idx_flat, embed_weight, n, d)
    return out.reshape(b, a * l, d)


# speedup vs baseline: 3.0053x; 3.0053x over previous
import functools

import jax
import jax.numpy as jnp
from jax import lax
from jax.experimental import pallas as pl
from jax.experimental.pallas import tpu as pltpu
from jax.experimental.pallas import tpu_sc as plsc

_NC = 2
_NS = 16
_NW = _NC * _NS

_SUB = 128


@functools.partial(jax.jit, static_argnames=("n", "d"))
def _sc_gather(idx_flat, table, n, d):
    v = table.shape[0]
    n_per_w = n // _NW
    n_sub = n_per_w // _SUB

    mesh = plsc.VectorSubcoreMesh(core_axis_name="c", subcore_axis_name="s")

    @functools.partial(
        pl.kernel,
        out_type=jax.ShapeDtypeStruct((n, d), jnp.float32),
        mesh=mesh,
        scratch_types=[
            pltpu.VMEM((n_per_w,), jnp.int32),
            pltpu.VMEM_SHARED((v, d), jnp.float32),
            pltpu.VMEM((_SUB, d), jnp.float32),
            pltpu.VMEM((_SUB, d), jnp.float32),
            pltpu.SemaphoreType.DMA,
            pltpu.SemaphoreType.DMA,
            pltpu.SemaphoreType.DMA,
            pltpu.SemaphoreType.DMA,
        ],
    )
    def k(idx_hbm, table_hbm, out_hbm, idx_v, table_v, rows0, rows1,
          g0, g1, w0, w1):
        wid = lax.axis_index("s") * _NC + lax.axis_index("c")
        base = wid * n_per_w

        @pl.when(lax.axis_index("s") == 0)
        def _():
            pltpu.sync_copy(table_hbm, table_v)

        plsc.subcore_barrier()
        pltpu.sync_copy(idx_hbm.at[pl.ds(base, n_per_w)], idx_v)

        def gather_start(j, rows, sem):
            pltpu.async_copy(table_v.at[idx_v.at[pl.ds(j * _SUB, _SUB)]], rows, sem)

        def gather_wait(rows, sem):
            pltpu.make_async_copy(table_v.at[idx_v.at[pl.ds(0, _SUB)]], rows, sem).wait()

        def write_start(j, rows, sem):
            pltpu.async_copy(rows, out_hbm.at[pl.ds(base + j * _SUB, _SUB)], sem)

        def write_wait(rows, sem):
            pltpu.make_async_copy(rows, out_hbm.at[pl.ds(base, _SUB)], sem).wait()

        gather_start(0, rows0, g0)
        n_pair = n_sub // 2

        def pair(i, carry):
            c = 2 * i

            @pl.when(i > 0)
            def _():
                write_wait(rows1, w1)

            gather_start(c + 1, rows1, g1)
            gather_wait(rows0, g0)
            write_start(c, rows0, w0)

            write_wait(rows0, w0)

            @pl.when(i + 1 < n_pair)
            def _():
                gather_start(c + 2, rows0, g0)

            gather_wait(rows1, g1)
            write_start(c + 1, rows1, w1)
            return carry

        lax.fori_loop(0, n_pair, pair, 0)
        write_wait(rows1, w1)

    return k(idx_flat, table)


@functools.partial(jax.jit, static_argnames=("n", "d"))
def _tc_onehot(idx_flat, table, n, d):
    blk = 2048
    tbl = jnp.zeros((128, d), table.dtype).at[: table.shape[0]].set(table)
    idx2 = idx_flat.reshape(n, 1)

    def body(idx_ref, tbl_ref, out_ref):
        oh = (idx_ref[...] == lax.broadcasted_iota(jnp.int32, (blk, 128), 1))
        out_ref[...] = jnp.dot(oh.astype(jnp.float32), tbl_ref[...],
                               preferred_element_type=jnp.float32)

    return pl.pallas_call(
        body,
        grid=(n // blk,),
        in_specs=[pl.BlockSpec((blk, 1), lambda i: (i, 0)),
                  pl.BlockSpec((128, d), lambda i: (0, 0))],
        out_specs=pl.BlockSpec((blk, d), lambda i: (i, 0)),
        out_shape=jax.ShapeDtypeStruct((n, d), jnp.float32),
    )(idx2, tbl)


def kernel(actions, embed_weight):
    b, a, l = actions.shape
    v, d = embed_weight.shape
    n = b * a * l
    idx_flat = actions.reshape(n)
    out = _sc_gather(idx_flat, embed_weight, n, d)
    return out.reshape(b, a * l, d)
